# baseline (speedup 1.0000x reference)
import functools

import jax
import jax.numpy as jnp
from jax import lax
from jax.experimental import pallas as pl
from jax.experimental.pallas import tpu as pltpu
from jax.experimental.pallas import tpu_sc as plsc

N = 10000
D = 128
E = 320000

NC = 2
NS = 16
NW = NC * NS

CHUNK = 128
CPT = 80
E_PAD = NW * CPT * CHUNK
N_PAD = 10112
RPT = N_PAD // NS
CPT_FAST = 160
CID_FAST = 1
_MESH = plsc.VectorSubcoreMesh(core_axis_name="c", subcore_axis_name="s")


_DEG_GRP = 16


@functools.partial(
    pl.kernel,
    out_type=jax.ShapeDtypeStruct((NC, N_PAD, D), jnp.float32),
    mesh=_MESH,
    scratch_types=[
        pltpu.VMEM((CPT, CHUNK), jnp.int32),
        pltpu.VMEM((CHUNK, D), jnp.float32),
        pltpu.VMEM_SHARED((N_PAD, D), jnp.float32),
        pltpu.SemaphoreType.DMA,
    ],
)
def _sc_degree(dst_hbm, ones_hbm, zrows_hbm, dp_hbm, dst_v, ones_v, degacc, sem):
    cid = lax.axis_index("c")
    sid = lax.axis_index("s")
    w = cid * NS + sid
    pltpu.sync_copy(zrows_hbm, degacc.at[pl.ds(sid * RPT, RPT)])
    pltpu.sync_copy(ones_hbm, ones_v)
    pltpu.sync_copy(dst_hbm.at[pl.ds(w * CPT, CPT)], dst_v)
    plsc.subcore_barrier()

    def group(gi, carry):
        def fire(k, c):
            pltpu.async_copy(ones_v, degacc.at[dst_v.at[gi * _DEG_GRP + k]],
                             sem, add=True)
            return c

        def drain(k, c):
            pltpu.make_async_copy(
                ones_v, degacc.at[dst_v.at[gi * _DEG_GRP + k]], sem).wait()
            return c

        lax.fori_loop(0, _DEG_GRP, fire, carry)
        lax.fori_loop(0, _DEG_GRP, drain, carry)
        return carry

    lax.fori_loop(0, CPT // _DEG_GRP, group, 0)
    plsc.subcore_barrier()
    pltpu.sync_copy(degacc.at[pl.ds(sid * RPT, RPT)],
                    dp_hbm.at[cid, pl.ds(sid * RPT, RPT)])


@functools.partial(
    pl.kernel,
    out_type=jax.ShapeDtypeStruct((N_PAD, D), jnp.float32),
    mesh=_MESH,
    scratch_types=[
        pltpu.VMEM((32, CHUNK), jnp.int32),
        pltpu.VMEM((32, CHUNK), jnp.int32),
        pltpu.VMEM((CHUNK, D), jnp.float32),
        pltpu.VMEM((CHUNK, D), jnp.float32),
        pltpu.VMEM_SHARED((N_PAD, D), jnp.float32),
        pltpu.SemaphoreType.DMA,
        pltpu.SemaphoreType.DMA,
    ],
)
def _sc_scatter(g_hbm, src_hbm, dst_hbm, zrows_hbm, out_hbm,
                src_v, dst_v, rows_a, rows_b, acc, sem_a, sem_b):
    cid = lax.axis_index("c")
    sid = lax.axis_index("s")
    pltpu.sync_copy(zrows_hbm, acc.at[pl.ds(sid * RPT, RPT)])
    plsc.subcore_barrier()

    def edge_pass(base, n_chunks, carry):
        pltpu.sync_copy(src_hbm.at[pl.ds(base, n_chunks)],
                        src_v.at[pl.ds(0, n_chunks)])
        pltpu.sync_copy(dst_hbm.at[pl.ds(base, n_chunks)],
                        dst_v.at[pl.ds(0, n_chunks)])
        nh = n_chunks // 2
        pltpu.async_copy(g_hbm.at[src_v.at[0]], rows_a, sem_a)

        def body(j, c):
            ia = 2 * j
            ib = ia + 1
            pltpu.async_copy(g_hbm.at[src_v.at[ib]], rows_b, sem_b)
            pltpu.make_async_copy(g_hbm.at[src_v.at[ia]], rows_a, sem_a).wait()
            pltpu.sync_copy(rows_a, acc.at[dst_v.at[ia]], add=True)

            @pl.when(j < nh - 1)
            def _():
                pltpu.async_copy(g_hbm.at[src_v.at[ia + 2]], rows_a, sem_a)

            pltpu.make_async_copy(g_hbm.at[src_v.at[ib]], rows_b, sem_b).wait()
            pltpu.sync_copy(rows_b, acc.at[dst_v.at[ib]], add=True)
            return c

        return lax.fori_loop(0, nh, body, carry)

    @pl.when(cid == CID_FAST)
    def _():
        lax.fori_loop(0, CPT_FAST // 32,
                      lambda h, c: edge_pass(sid * CPT_FAST + h * 32, 32, c), 0)

    plsc.subcore_barrier()

    @pl.when(cid == CID_FAST)
    def _():
        pltpu.sync_copy(acc.at[pl.ds(sid * RPT, RPT)],
                        out_hbm.at[pl.ds(sid * RPT, RPT)])


_BLK = N_PAD // 8


def _dis_of(dp_blk):
    deg = dp_blk[0, :, 0] + dp_blk[1, :, 0] + 1.0
    return lax.rsqrt(deg)


def _tc1_body(x_ref, w_ref, dp_ref, o_ref):
    dis = _dis_of(dp_ref[...])
    h = jnp.dot(x_ref[...], w_ref[...], preferred_element_type=jnp.float32)
    o_ref[...] = h * dis[:, None]


def _tc2_body(p_ref, g_ref, dp_ref, b_ref, w_ref, o_ref):
    dis = _dis_of(dp_ref[...])
    u = dis[:, None] * (p_ref[...] + g_ref[...]) + b_ref[...]
    t = jnp.where(u >= 0, u, 0.01 * u)
    h = jnp.dot(t, w_ref[...], preferred_element_type=jnp.float32)
    o_ref[...] = h * dis[:, None]


def _tc3_body(p_ref, g_ref, dp_ref, b_ref, o_ref):
    dis = _dis_of(dp_ref[...])
    u = dis[:, None] * (p_ref[...] + g_ref[...]) + b_ref[...]
    o_ref[...] = jnp.where(u >= 0, u, 0.01 * u)


_rows_spec = pl.BlockSpec((_BLK, D), lambda i: (i, 0))
_w_spec = pl.BlockSpec((D, D), lambda i: (0, 0))
_dp_spec = pl.BlockSpec((NC, _BLK, D), lambda i: (0, i, 0))
_p_spec = pl.BlockSpec((NC, _BLK, D), lambda i: (0, i, 0))
_b_spec = pl.BlockSpec((1, D), lambda i: (0, 0))
_out_shape = jax.ShapeDtypeStruct((N_PAD, D), jnp.float32)

_tc1 = pl.pallas_call(
    _tc1_body, grid=(8,),
    in_specs=[_rows_spec, _w_spec, _dp_spec],
    out_specs=_rows_spec, out_shape=_out_shape)

_tc2 = pl.pallas_call(
    _tc2_body, grid=(8,),
    in_specs=[_rows_spec, _rows_spec, _dp_spec, _b_spec, _w_spec],
    out_specs=_rows_spec, out_shape=_out_shape)

_tc3 = pl.pallas_call(
    _tc3_body, grid=(8,),
    in_specs=[_rows_spec, _rows_spec, _dp_spec, _b_spec],
    out_specs=_rows_spec, out_shape=_out_shape)


def kernel(node_features, edge_index, W1, b1, W2, b2):
    pad = jnp.full((E_PAD - E,), N, dtype=jnp.int32)
    src = jnp.concatenate([edge_index[0].astype(jnp.int32), pad])
    dst = jnp.concatenate([edge_index[1].astype(jnp.int32), pad])
    src = src.reshape(E_PAD // CHUNK, CHUNK)
    dst = dst.reshape(E_PAD // CHUNK, CHUNK)

    x_pad = jnp.pad(node_features, ((0, N_PAD - N), (0, 0)))
    ones_deg = jnp.ones((CHUNK, D), jnp.float32)
    zrows = jnp.zeros((RPT, D), jnp.float32)
    b1r = b1.reshape(1, D)
    b2r = b2.reshape(1, D)

    dp = _sc_degree(dst, ones_deg, zrows)
    g1 = _tc1(x_pad, W1, dp)
    p1 = _sc_scatter(g1, src, dst, zrows)
    g2 = _tc2(p1, g1, dp, b1r, W2)
    p2 = _sc_scatter(g2, src, dst, zrows)
    out = _tc3(p2, g2, dp, b2r)
    return out[:N]


# speedup vs baseline: 3.0577x; 3.0577x over previous
import functools

import jax
import jax.numpy as jnp
from jax import lax
from jax.experimental import pallas as pl
from jax.experimental.pallas import tpu as pltpu
from jax.experimental.pallas import tpu_sc as plsc

N = 10000
D = 128
E = 320000

NC = 2
NS = 16
NW = NC * NS

CHUNK = 128
CPT = 80
E_PAD = NW * CPT * CHUNK
N_PAD = 10112
RPT = N_PAD // NS
_MESH = plsc.VectorSubcoreMesh(core_axis_name="c", subcore_axis_name="s")


_DEG_GRP = 16


@functools.partial(
    pl.kernel,
    out_type=jax.ShapeDtypeStruct((NC, N_PAD, D), jnp.float32),
    mesh=_MESH,
    scratch_types=[
        pltpu.VMEM((CPT, CHUNK), jnp.int32),
        pltpu.VMEM((CHUNK, D), jnp.float32),
        pltpu.VMEM_SHARED((N_PAD, D), jnp.float32),
        pltpu.SemaphoreType.DMA,
    ],
)
def _sc_degree(dst_hbm, ones_hbm, zrows_hbm, dp_hbm, dst_v, ones_v, degacc, sem):
    cid = lax.axis_index("c")
    sid = lax.axis_index("s")
    w = cid * NS + sid
    pltpu.sync_copy(zrows_hbm, degacc.at[pl.ds(sid * RPT, RPT)])
    pltpu.sync_copy(ones_hbm, ones_v)
    pltpu.sync_copy(dst_hbm.at[pl.ds(w * CPT, CPT)], dst_v)
    plsc.subcore_barrier()

    def group(gi, carry):
        def fire(k, c):
            pltpu.async_copy(ones_v, degacc.at[dst_v.at[gi * _DEG_GRP + k]],
                             sem, add=True)
            return c

        def drain(k, c):
            pltpu.make_async_copy(
                ones_v, degacc.at[dst_v.at[gi * _DEG_GRP + k]], sem).wait()
            return c

        lax.fori_loop(0, _DEG_GRP, fire, carry)
        lax.fori_loop(0, _DEG_GRP, drain, carry)
        return carry

    lax.fori_loop(0, CPT // _DEG_GRP, group, 0)
    plsc.subcore_barrier()
    pltpu.sync_copy(degacc.at[pl.ds(sid * RPT, RPT)],
                    dp_hbm.at[cid, pl.ds(sid * RPT, RPT)])


@functools.partial(
    pl.kernel,
    out_type=jax.ShapeDtypeStruct((NC, N_PAD, D), jnp.float32),
    mesh=_MESH,
    scratch_types=[
        pltpu.VMEM((40, CHUNK), jnp.int32),
        pltpu.VMEM((40, CHUNK), jnp.int32),
        pltpu.VMEM((CHUNK, D), jnp.float32),
        pltpu.VMEM((CHUNK, D), jnp.float32),
        pltpu.VMEM_SHARED((N_PAD, D), jnp.float32),
        pltpu.SemaphoreType.DMA,
        pltpu.SemaphoreType.DMA,
    ],
)
def _sc_scatter(g_hbm, src_hbm, dst_hbm, zrows_hbm, out_hbm,
                src_v, dst_v, rows_a, rows_b, acc, sem_a, sem_b):
    cid = lax.axis_index("c")
    sid = lax.axis_index("s")
    pltpu.sync_copy(zrows_hbm, acc.at[pl.ds(sid * RPT, RPT)])
    plsc.subcore_barrier()

    def edge_pass(base, n_chunks, carry):
        pltpu.sync_copy(src_hbm.at[pl.ds(base, n_chunks)],
                        src_v.at[pl.ds(0, n_chunks)])
        pltpu.sync_copy(dst_hbm.at[pl.ds(base, n_chunks)],
                        dst_v.at[pl.ds(0, n_chunks)])
        nh = n_chunks // 2
        pltpu.async_copy(g_hbm.at[src_v.at[0]], rows_a, sem_a)

        def body(j, c):
            ia = 2 * j
            ib = ia + 1
            pltpu.async_copy(g_hbm.at[src_v.at[ib]], rows_b, sem_b)
            pltpu.make_async_copy(g_hbm.at[src_v.at[ia]], rows_a, sem_a).wait()
            pltpu.sync_copy(rows_a, acc.at[dst_v.at[ia]], add=True)

            @pl.when(j < nh - 1)
            def _():
                pltpu.async_copy(g_hbm.at[src_v.at[ia + 2]], rows_a, sem_a)

            pltpu.make_async_copy(g_hbm.at[src_v.at[ib]], rows_b, sem_b).wait()
            pltpu.sync_copy(rows_b, acc.at[dst_v.at[ib]], add=True)
            return c

        return lax.fori_loop(0, nh, body, carry)

    w = cid * NS + sid
    lax.fori_loop(0, 2, lambda h, c: edge_pass(w * CPT + h * 40, 40, c), 0)

    plsc.subcore_barrier()
    pltpu.sync_copy(acc.at[pl.ds(sid * RPT, RPT)],
                    out_hbm.at[cid, pl.ds(sid * RPT, RPT)])


_BLK = N_PAD // 8


def _dis_of(dp_blk):
    deg = dp_blk[0, :, 0] + dp_blk[1, :, 0] + 1.0
    return lax.rsqrt(deg)


def _tc1_body(x_ref, w_ref, dp_ref, o_ref):
    dis = _dis_of(dp_ref[...])
    h = jnp.dot(x_ref[...], w_ref[...], preferred_element_type=jnp.float32)
    o_ref[...] = h * dis[:, None]


def _tc2_body(p_ref, g_ref, dp_ref, b_ref, w_ref, o_ref):
    dis = _dis_of(dp_ref[...])
    p = p_ref[...]
    u = dis[:, None] * (p[0] + p[1] + g_ref[...]) + b_ref[...]
    t = jnp.where(u >= 0, u, 0.01 * u)
    h = jnp.dot(t, w_ref[...], preferred_element_type=jnp.float32)
    o_ref[...] = h * dis[:, None]


def _tc3_body(p_ref, g_ref, dp_ref, b_ref, o_ref):
    dis = _dis_of(dp_ref[...])
    p = p_ref[...]
    u = dis[:, None] * (p[0] + p[1] + g_ref[...]) + b_ref[...]
    o_ref[...] = jnp.where(u >= 0, u, 0.01 * u)


_rows_spec = pl.BlockSpec((_BLK, D), lambda i: (i, 0))
_w_spec = pl.BlockSpec((D, D), lambda i: (0, 0))
_dp_spec = pl.BlockSpec((NC, _BLK, D), lambda i: (0, i, 0))
_p_spec = pl.BlockSpec((NC, _BLK, D), lambda i: (0, i, 0))
_b_spec = pl.BlockSpec((1, D), lambda i: (0, 0))
_out_shape = jax.ShapeDtypeStruct((N_PAD, D), jnp.float32)

_tc1 = pl.pallas_call(
    _tc1_body, grid=(8,),
    in_specs=[_rows_spec, _w_spec, _dp_spec],
    out_specs=_rows_spec, out_shape=_out_shape)

_tc2 = pl.pallas_call(
    _tc2_body, grid=(8,),
    in_specs=[_p_spec, _rows_spec, _dp_spec, _b_spec, _w_spec],
    out_specs=_rows_spec, out_shape=_out_shape)

_tc3 = pl.pallas_call(
    _tc3_body, grid=(8,),
    in_specs=[_p_spec, _rows_spec, _dp_spec, _b_spec],
    out_specs=_rows_spec, out_shape=_out_shape)


def kernel(node_features, edge_index, W1, b1, W2, b2):
    seq = jnp.arange(E_PAD - E, dtype=jnp.int32)
    src = jnp.concatenate([edge_index[0].astype(jnp.int32), seq % N])
    dst = jnp.concatenate([edge_index[1].astype(jnp.int32),
                           N + seq % (N_PAD - N)])
    src = src.reshape(E_PAD // CHUNK, CHUNK)
    dst = dst.reshape(E_PAD // CHUNK, CHUNK)

    x_pad = jnp.pad(node_features, ((0, N_PAD - N), (0, 0)))
    ones_deg = jnp.ones((CHUNK, D), jnp.float32)
    zrows = jnp.zeros((RPT, D), jnp.float32)
    b1r = b1.reshape(1, D)
    b2r = b2.reshape(1, D)

    dp = _sc_degree(dst, ones_deg, zrows)
    g1 = _tc1(x_pad, W1, dp)
    p1 = _sc_scatter(g1, src, dst, zrows)
    g2 = _tc2(p1, g1, dp, b1r, W2)
    p2 = _sc_scatter(g2, src, dst, zrows)
    out = _tc3(p2, g2, dp, b2r)
    return out[:N]
